# Initial kernel scaffold; baseline (speedup 1.0000x reference)
import functools

import jax
import jax.numpy as jnp
from jax import lax
from jax.experimental import pallas as pl
from jax.experimental.pallas import tpu as pltpu
from jax.experimental.pallas import tpu_sc as plsc

_NC = 2
_NS = 16
_NW = _NC * _NS


def _make_sc_expand(S, D):
    rows = 2 * S
    n_chunks = rows // 128
    slices_per_w = (2 * S) // _NW

    mesh = plsc.VectorSubcoreMesh(core_axis_name="c", subcore_axis_name="s")

    @functools.partial(
        pl.kernel,
        mesh=mesh,
        out_type=jax.ShapeDtypeStruct((2 * S, S, D), jnp.float32),
        scratch_types=[
            pltpu.VMEM((rows // 128, 128), jnp.int32),
            pltpu.VMEM((rows, D), jnp.float32),
            pltpu.SemaphoreType.DMA,
        ],
    )
    def expand(table_hbm, idx_hbm, out_hbm, idx_v, flip_v, sem):
        cid = lax.axis_index("c")
        sid = lax.axis_index("s")
        wid = sid * _NC + cid

        pltpu.sync_copy(idx_hbm, idx_v)
        gathers = [
            pltpu.async_copy(
                table_hbm.at[idx_v.at[r]],
                flip_v.at[pl.ds(r * 128, 128)],
                sem,
            )
            for r in range(n_chunks)
        ]
        for g in gathers:
            g.wait()

        base = wid * slices_per_w
        i0 = lax.rem(base, S)
        copies = []
        for t in range(slices_per_w):
            off = (S - 1) - (i0 + t)
            copies.append(
                pltpu.async_copy(
                    flip_v.at[pl.ds(off, S)],
                    out_hbm.at[base + t],
                    sem,
                )
            )
        for cp in copies:
            cp.wait()

    return expand


def kernel(rel_pos_embedding, batch_size, seq_len):
    n_rows, D = rel_pos_embedding.shape
    S = (n_rows + 1) // 2
    static_batch = 2

    shift = (seq_len - S) + (batch_size - static_batch)
    k = jnp.arange(2 * S, dtype=jnp.int32)
    idx = jnp.clip((2 * S - 2) - k + shift, 0, n_rows - 1).astype(jnp.int32)
    idx = idx.reshape((2 * S) // 128, 128)

    out = _make_sc_expand(S, D)(rel_pos_embedding, idx)
    return out.reshape(static_batch, S, S, D)


# speedup vs baseline: 8.5924x; 8.5924x over previous
import functools

import jax
import jax.numpy as jnp
from jax import lax
from jax.experimental import pallas as pl
from jax.experimental.pallas import tpu as pltpu
from jax.experimental.pallas import tpu_sc as plsc

_NC = 2
_NS = 16
_NW = _NC * _NS
_L = 16


def _make_sc_expand(S, D):
    rows = 2 * S - 1
    slices_per_w = (2 * S) // _NW
    mesh = plsc.VectorSubcoreMesh(core_axis_name="c", subcore_axis_name="s")

    @functools.partial(
        pl.kernel,
        mesh=mesh,
        out_type=jax.ShapeDtypeStruct((2 * S, S, D), jnp.float32),
        scratch_types=[
            pltpu.VMEM((2 * S, D), jnp.float32),
            pltpu.SemaphoreType.DMA,
        ],
    )
    def expand(table_hbm, out_hbm, buf, sem):
        cid = lax.axis_index("c")
        sid = lax.axis_index("s")
        wid = sid * _NC + cid

        pltpu.sync_copy(table_hbm, buf.at[pl.ds(0, rows)])

        def swap_rows(k, _):
            lo = k
            hi = (rows - 1) - k
            for q in range(D // _L):
                a = buf[lo, pl.ds(q * _L, _L)]
                b = buf[hi, pl.ds(q * _L, _L)]
                buf[lo, pl.ds(q * _L, _L)] = b
                buf[hi, pl.ds(q * _L, _L)] = a
            return 0

        lax.fori_loop(0, (rows - 1) // 2, swap_rows, 0)

        base = wid * slices_per_w
        i0 = lax.rem(base, S)
        copies = []
        for t in range(slices_per_w):
            off = (S - 1) - (i0 + t)
            copies.append(
                pltpu.async_copy(
                    buf.at[pl.ds(off, S)],
                    out_hbm.at[base + t],
                    sem,
                )
            )
        for cp in copies:
            cp.wait()

    return expand


def kernel(rel_pos_embedding, batch_size, seq_len):
    n_rows, D = rel_pos_embedding.shape
    S = (n_rows + 1) // 2
    static_batch = 2

    shift = (seq_len - S) + (batch_size - static_batch)
    r = jnp.arange(n_rows, dtype=jnp.int32)
    table_adj = rel_pos_embedding[jnp.clip(r + shift, 0, n_rows - 1)]

    out = _make_sc_expand(S, D)(table_adj)
    return out.reshape(static_batch, S, S, D)
